# Initial kernel scaffold; baseline (speedup 1.0000x reference)
import jax
import jax.numpy as jnp
from jax.experimental import pallas as pl


def kernel(x, sources, targets, norm, W):
    raise NotImplementedError("write your pallas kernel here")


# speedup vs baseline: 6.6007x; 6.6007x over previous
import functools

import jax
import jax.numpy as jnp
from jax import lax
from jax.experimental import pallas as pl
from jax.experimental.pallas import tpu as pltpu
from jax.experimental.pallas import tpu_sc as plsc

N = 50000
C = 64
E = 800000

NSC = 2
NTILE = 16
HALF = N // NSC
ACC_ROWS = HALF + 8
DUMMY = HALF
EPT = E // NTILE
K = 80
B = 2000
NBLK = EPT // B
CPB = B // K
VPC = K // 16

STRIPE = 1568
LAST_STRIPE = HALF - (NTILE - 1) * STRIPE


def _sc_body(x_hbm, src_hbm, tgt_hbm, out_hbm,
             src_buf, tgt_buf, rows0, rows1, sidx0, sidx1, acc, sem0, sem1):
    sc = lax.axis_index("c")
    tile = lax.axis_index("s")
    node_base = sc * HALF

    @pl.when(tile < NTILE - 1)
    def _():
        r0 = tile * STRIPE
        pltpu.sync_copy(x_hbm.at[pl.ds(node_base + r0, STRIPE)],
                        acc.at[pl.ds(r0, STRIPE)])

    @pl.when(tile == NTILE - 1)
    def _():
        r0 = (NTILE - 1) * STRIPE
        pltpu.sync_copy(x_hbm.at[pl.ds(node_base + r0, LAST_STRIPE)],
                        acc.at[pl.ds(r0, LAST_STRIPE)])

    plsc.subcore_barrier()

    def _gather(j, rows, sem):
        pltpu.async_copy(x_hbm.at[src_buf.at[pl.ds(j * K, K)]], rows, sem)

    def _process(j, rows, sem, sidx):
        for u in range(VPC):
            t = tgt_buf[pl.ds(j * K + u * 16, 16)]
            lo = t - node_base
            ok = (lo >= 0) & (lo < HALF)
            sidx[pl.ds(u * 16, 16)] = jnp.where(ok, lo, DUMMY)
        pltpu.make_async_copy(
            x_hbm.at[src_buf.at[pl.ds(j * K, K)]], rows, sem).wait()
        pltpu.sync_copy(rows, acc.at[sidx], add=True)

        @pl.when(j + 2 < CPB)
        def _():
            _gather(j + 2, rows, sem)

    def _run_block(blk, carry):
        ebase = tile * EPT + blk * B
        pltpu.sync_copy(src_hbm.at[pl.ds(ebase, B)], src_buf)
        pltpu.sync_copy(tgt_hbm.at[pl.ds(ebase, B)], tgt_buf)

        _gather(0, rows0, sem0)
        _gather(1, rows1, sem1)

        def _pair(i, c):
            _process(i * 2, rows0, sem0, sidx0)
            _process(i * 2 + 1, rows1, sem1, sidx1)
            return c

        lax.fori_loop(0, CPB // 2, _pair, 0)
        _process(CPB - 1, rows0, sem0, sidx0)
        return carry

    lax.fori_loop(0, NBLK, _run_block, 0)

    plsc.subcore_barrier()

    @pl.when(tile < NTILE - 1)
    def _():
        r0 = tile * STRIPE
        pltpu.sync_copy(acc.at[pl.ds(r0, STRIPE)],
                        out_hbm.at[pl.ds(node_base + r0, STRIPE)])

    @pl.when(tile == NTILE - 1)
    def _():
        r0 = (NTILE - 1) * STRIPE
        pltpu.sync_copy(acc.at[pl.ds(r0, LAST_STRIPE)],
                        out_hbm.at[pl.ds(node_base + r0, LAST_STRIPE)])


_sc_aggregate = functools.partial(
    pl.kernel,
    out_type=jax.ShapeDtypeStruct((N, C), jnp.float32),
    mesh=plsc.VectorSubcoreMesh(core_axis_name="c", subcore_axis_name="s"),
    scratch_types=[
        pltpu.VMEM((B,), jnp.int32),
        pltpu.VMEM((B,), jnp.int32),
        pltpu.VMEM((K, C), jnp.float32),
        pltpu.VMEM((K, C), jnp.float32),
        pltpu.VMEM((K,), jnp.int32),
        pltpu.VMEM((K,), jnp.int32),
        pltpu.VMEM_SHARED((ACC_ROWS, C), jnp.float32),
        pltpu.SemaphoreType.DMA,
        pltpu.SemaphoreType.DMA,
    ],
    compiler_params=pltpu.CompilerParams(use_tc_tiling_on_sc=False),
)(_sc_body)


_R = 2000


def _tc_body(agg_ref, norm_ref, w_ref, out_ref):
    h = norm_ref[...] * agg_ref[...]
    out_ref[...] = lax.dot_general(
        h, w_ref[...], (((1,), (0,)), ((), ())),
        precision=lax.Precision.HIGHEST,
        preferred_element_type=jnp.float32)


def _tc_matmul(agg, norm, w):
    return pl.pallas_call(
        _tc_body,
        grid=(N // _R,),
        in_specs=[
            pl.BlockSpec((_R, C), lambda i: (i, 0)),
            pl.BlockSpec((_R, 1), lambda i: (i, 0)),
            pl.BlockSpec((C, C), lambda i: (0, 0)),
        ],
        out_specs=pl.BlockSpec((_R, C), lambda i: (i, 0)),
        out_shape=jax.ShapeDtypeStruct((N, C), jnp.float32),
    )(agg, norm, w)


def kernel(x, sources, targets, norm, W):
    src = sources.astype(jnp.int32)
    tgt = targets.astype(jnp.int32)
    agg = _sc_aggregate(x, src, tgt)
    return _tc_matmul(agg, norm, W)


# speedup vs baseline: 6.6870x; 1.0131x over previous
import functools

import jax
import jax.numpy as jnp
from jax import lax
from jax.experimental import pallas as pl
from jax.experimental.pallas import tpu as pltpu
from jax.experimental.pallas import tpu_sc as plsc

N = 50000
C = 64
E = 800000

NSC = 2
NTILE = 16
HALF = N // NSC
ACC_ROWS = HALF + 8
DUMMY = HALF
K = 80
CPB = 16
B = K * CPB
EPT = 49920
NBLK_LO = EPT // B
NBLK_HI = (E - (NTILE - 1) * EPT) // B
NBUF = 4
VPC = K // 16

STRIPE = 1568
LAST_STRIPE = HALF - (NTILE - 1) * STRIPE


def _sc_body(x_hbm, src_hbm, tgt_hbm, out_hbm,
             sbufs0, sbuft0, sbufs1, sbuft1,
             rows0, rows1, rows2, rows3,
             sidx0, sidx1, sidx2, sidx3,
             acc,
             semg0, semg1, semg2, semg3,
             sems0, sems1, sems2, sems3,
             semi0, semi1):
    sc = lax.axis_index("c")
    tile = lax.axis_index("s")
    node_base = sc * HALF
    ebase = tile * EPT
    nblk = jnp.where(tile == NTILE - 1, NBLK_HI, NBLK_LO)

    sbufs = (sbufs0, sbufs1)
    sbuft = (sbuft0, sbuft1)
    semi = (semi0, semi1)
    rows = (rows0, rows1, rows2, rows3)
    sidx = (sidx0, sidx1, sidx2, sidx3)
    semg = (semg0, semg1, semg2, semg3)
    sems = (sems0, sems1, sems2, sems3)

    def _stage_start(blk, par):
        pltpu.async_copy(src_hbm.at[pl.ds(ebase + blk * B, B)],
                         sbufs[par], semi[par])
        pltpu.async_copy(tgt_hbm.at[pl.ds(ebase + blk * B, B)],
                         sbuft[par], semi[par])

    def _stage_wait(blk, par):
        pltpu.make_async_copy(src_hbm.at[pl.ds(ebase + blk * B, B)],
                              sbufs[par], semi[par]).wait()
        pltpu.make_async_copy(tgt_hbm.at[pl.ds(ebase + blk * B, B)],
                              sbuft[par], semi[par]).wait()

    def _gather_start(j, b, par):
        pltpu.async_copy(x_hbm.at[sbufs[par].at[pl.ds(j * K, K)]],
                         rows[b], semg[b])

    def _gather_wait(j, b, par):
        pltpu.make_async_copy(x_hbm.at[sbufs[par].at[pl.ds(j * K, K)]],
                              rows[b], semg[b]).wait()

    def _scatter_start(b):
        pltpu.async_copy(rows[b], acc.at[sidx[b]], sems[b], add=True)

    def _scatter_wait(b):
        pltpu.make_async_copy(rows[b], acc.at[sidx[b]], sems[b]).wait()

    _stage_start(0, 0)

    @pl.when(tile < NTILE - 1)
    def _():
        r0 = tile * STRIPE
        pltpu.sync_copy(x_hbm.at[pl.ds(node_base + r0, STRIPE)],
                        acc.at[pl.ds(r0, STRIPE)])

    @pl.when(tile == NTILE - 1)
    def _():
        r0 = (NTILE - 1) * STRIPE
        pltpu.sync_copy(x_hbm.at[pl.ds(node_base + r0, LAST_STRIPE)],
                        acc.at[pl.ds(r0, LAST_STRIPE)])

    plsc.subcore_barrier()

    def _run_block(blk, par):
        for jj in range(NBUF - 1):
            _gather_start(jj, jj, par)

        def _grp(gidx, c):
            for u in range(NBUF):
                j = gidx * NBUF + u
                bn = (u + NBUF - 1) % NBUF
                _gather_wait(j, u, par)
                for v in range(VPC):
                    t = sbuft[par][pl.ds(j * K + v * 16, 16)]
                    lo = t - node_base
                    ok = (lo >= 0) & (lo < HALF)
                    sidx[u][pl.ds(v * 16, 16)] = jnp.where(ok, lo, DUMMY)
                _scatter_start(u)
                if u == 0:
                    @pl.when(gidx > 0)
                    def _():
                        _scatter_wait(bn)
                else:
                    _scatter_wait(bn)

                @pl.when(j < CPB - NBUF + 1)
                def _():
                    _gather_start(j + NBUF - 1, bn, par)
            return c

        lax.fori_loop(0, CPB // NBUF, _grp, 0)
        _scatter_wait(NBUF - 1)

    def _block_pair(p, carry):
        for par in range(2):
            blk = p * 2 + par

            @pl.when(blk < nblk)
            def _():
                _stage_wait(blk, par)

                @pl.when(blk + 1 < nblk)
                def _():
                    _stage_start(blk + 1, 1 - par)

                _run_block(blk, par)
        return carry

    lax.fori_loop(0, (NBLK_HI + 1) // 2, _block_pair, 0)

    plsc.subcore_barrier()

    @pl.when(tile < NTILE - 1)
    def _():
        r0 = tile * STRIPE
        pltpu.sync_copy(acc.at[pl.ds(r0, STRIPE)],
                        out_hbm.at[pl.ds(node_base + r0, STRIPE)])

    @pl.when(tile == NTILE - 1)
    def _():
        r0 = (NTILE - 1) * STRIPE
        pltpu.sync_copy(acc.at[pl.ds(r0, LAST_STRIPE)],
                        out_hbm.at[pl.ds(node_base + r0, LAST_STRIPE)])


_sc_aggregate = functools.partial(
    pl.kernel,
    out_type=jax.ShapeDtypeStruct((N, C), jnp.float32),
    mesh=plsc.VectorSubcoreMesh(core_axis_name="c", subcore_axis_name="s"),
    scratch_types=(
        [pltpu.VMEM((B,), jnp.int32)] * 4
        + [pltpu.VMEM((K, C), jnp.float32)] * 4
        + [pltpu.VMEM((K,), jnp.int32)] * 4
        + [pltpu.VMEM_SHARED((ACC_ROWS, C), jnp.float32)]
        + [pltpu.SemaphoreType.DMA] * 10
    ),
    compiler_params=pltpu.CompilerParams(use_tc_tiling_on_sc=False),
)(_sc_body)


_R = 2000


def _tc_body(agg_ref, norm_ref, w_ref, out_ref):
    h = norm_ref[...] * agg_ref[...]
    out_ref[...] = lax.dot_general(
        h, w_ref[...], (((1,), (0,)), ((), ())),
        precision=lax.Precision.HIGHEST,
        preferred_element_type=jnp.float32)


def _tc_matmul(agg, norm, w):
    return pl.pallas_call(
        _tc_body,
        grid=(N // _R,),
        in_specs=[
            pl.BlockSpec((_R, C), lambda i: (i, 0)),
            pl.BlockSpec((_R, 1), lambda i: (i, 0)),
            pl.BlockSpec((C, C), lambda i: (0, 0)),
        ],
        out_specs=pl.BlockSpec((_R, C), lambda i: (i, 0)),
        out_shape=jax.ShapeDtypeStruct((N, C), jnp.float32),
    )(agg, norm, w)


def kernel(x, sources, targets, norm, W):
    src = sources.astype(jnp.int32)
    tgt = targets.astype(jnp.int32)
    agg = _sc_aggregate(x, src, tgt)
    return _tc_matmul(agg, norm, W)
